# baseline (speedup 1.0000x reference)
import dataclasses
import functools

import jax
import jax.numpy as jnp
from jax import lax
from jax.experimental import pallas as pl
from jax.experimental.pallas import tpu as pltpu
from jax.experimental.pallas import tpu_sc as plsc


def kernel(x, edge_index, W, b):
    N, D = x.shape
    E = edge_index.shape[1]
    Dh = D // 2

    CH = 128
    NW = 16
    PAD_UNIT = 2 * NW * CH
    E_pad = ((E + PAD_UNIT - 1) // PAD_UNIT) * PAD_UNIT
    ROWS = E_pad // CH
    RPC = ROWS // 2
    RPS_A = RPC // NW
    RPS_B = ROWS // NW

    NT = N + NW
    ZB = (NT // (8 * NW)) * 8
    ZTAIL = NT - NW * ZB
    CB = (N // (8 * NW)) * 8
    CTAIL = N - NW * CB
    assert D % 2 == 0 and N % 8 == 0 and ZTAIL <= CH and CTAIL <= CH

    src = edge_index[0].astype(jnp.int32)
    dst = edge_index[1].astype(jnp.int32)
    pad = E_pad - E
    src_p = jnp.concatenate([src, jnp.zeros((pad,), jnp.int32)])
    dst_p = jnp.concatenate([dst, jnp.full((pad,), N, jnp.int32)])
    dst2d = dst_p.reshape(ROWS, CH)
    src2d = src_p.reshape(ROWS, CH)
    src3d = jnp.stack([src2d, src2d + N])

    zeros_b = jnp.zeros((CH, Dh), jnp.float32)

    mesh = plsc.VectorSubcoreMesh(core_axis_name="c", subcore_axis_name="s")


    EPW = E_pad // (2 * NW)
    assert EPW % 16 == 0 and N % 16 == 0

    @functools.partial(
        pl.kernel,
        out_type=jax.ShapeDtypeStruct((2, NW, 1, N), jnp.int32),
        mesh=mesh,
        scratch_types=[
            pltpu.VMEM((EPW,), jnp.int32),
            pltpu.VMEM((NT,), jnp.int32),
            pltpu.SemaphoreType.DMA,
        ],
        compiler_params=dataclasses.replace(
            pltpu.CompilerParams(), needs_layout_passes=False),
    )
    def degree_k(dst1d_hbm, deg_hbm, idx_v, hist_v, sem):
        c = lax.axis_index("c")
        s = lax.axis_index("s")
        w = c * NW + s
        pltpu.sync_copy(dst1d_hbm.at[pl.ds(w * EPW, EPW)], idx_v)

        @pl.loop(0, NT, step=16)
        def _(i):
            hist_v[pl.ds(i, 16)] = jnp.zeros((16,), jnp.int32)

        ones16 = jnp.ones((16,), jnp.int32)

        @pl.loop(0, EPW, step=16)
        def _(i):
            plsc.addupdate_scatter(hist_v, [idx_v[pl.ds(i, 16)]], ones16)

        pltpu.sync_copy(hist_v.at[pl.ds(0, N)], deg_hbm.at[c, s, 0])

    deg_hists = degree_k(dst_p)

    def degsum_body(d_ref, o_ref):
        o_ref[...] = jnp.sum(d_ref[...], axis=(0, 1))

    cnt_row = pl.pallas_call(
        degsum_body,
        grid=(1,),
        in_specs=[pl.BlockSpec((2, NW, 1, N), lambda i: (0, 0, 0, 0))],
        out_specs=pl.BlockSpec((1, N), lambda i: (0, 0)),
        out_shape=jax.ShapeDtypeStruct((1, N), jnp.int32),
    )(deg_hists)
    cnt_col = cnt_row.reshape(N, 1)

    BM = 1000

    def mm_body(x_ref, w_ref, o_ref):
        o_ref[...] = jnp.dot(x_ref[...], w_ref[...],
                             preferred_element_type=jnp.float32)

    h = pl.pallas_call(
        mm_body,
        grid=(N // BM,),
        in_specs=[pl.BlockSpec((BM, D), lambda r: (r, 0)),
                  pl.BlockSpec((D, D), lambda r: (0, 0))],
        out_specs=pl.BlockSpec((BM, D), lambda r: (r, 0)),
        out_shape=jax.ShapeDtypeStruct((N, D), jnp.float32),
    )(x, W)

    def scale_body(h_ref, cnt_ref, o_ref):
        dis = lax.rsqrt(cnt_ref[...].astype(jnp.float32) + 1.0)
        o_ref[...] = h_ref[...] * dis

    h2s = pl.pallas_call(
        scale_body,
        grid=(N // BM, 2),
        in_specs=[pl.BlockSpec((BM, Dh), lambda r, c: (r, c)),
                  pl.BlockSpec((BM, 1), lambda r, c: (r, 0))],
        out_specs=pl.BlockSpec((BM, Dh), lambda r, c: (c * (N // BM) + r, 0)),
        out_shape=jax.ShapeDtypeStruct((2 * N, Dh), jnp.float32),
    )(h, cnt_col)

    HALF = RPS_B // 2
    assert HALF % 2 == 0

    @functools.partial(
        pl.kernel,
        out_type=jax.ShapeDtypeStruct((2, N, Dh), jnp.float32),
        mesh=mesh,
        scratch_types=[
            pltpu.VMEM((HALF, CH), jnp.int32),
            pltpu.VMEM((HALF, CH), jnp.int32),
            pltpu.VMEM((CH, Dh), jnp.float32),
            pltpu.VMEM((CH, Dh), jnp.float32),
            pltpu.VMEM_SHARED((NT, Dh), jnp.float32),
            pltpu.SemaphoreType.DMA,
            pltpu.SemaphoreType.DMA,
            pltpu.SemaphoreType.DMA,
            pltpu.SemaphoreType.DMA,
        ],
    )
    def agg_k(src_hbm, dst_hbm, tab_hbm, zeros_hbm, out_hbm,
              src_v, dst_v, rows0, rows1, acc_sh,
              sem_g0, sem_g1, sem_s0, sem_s1):
        c = lax.axis_index("c")
        s = lax.axis_index("s")
        pltpu.sync_copy(zeros_hbm, rows0)
        zbase = s * ZB
        nfull = ZB // CH

        @pl.loop(0, nfull)
        def _(i):
            pltpu.sync_copy(rows0, acc_sh.at[pl.ds(zbase + i * CH, CH)])

        rem = ZB - nfull * CH
        pltpu.sync_copy(rows0.at[pl.ds(0, rem)],
                        acc_sh.at[pl.ds(zbase + nfull * CH, rem)])

        @pl.when(s == NW - 1)
        def _():
            pltpu.sync_copy(rows0.at[pl.ds(0, ZTAIL)],
                            acc_sh.at[pl.ds(NW * ZB, ZTAIL)])

        plsc.subcore_barrier()
        row0 = s * RPS_B

        for half in range(2):
            hbase = row0 + half * HALF
            pltpu.sync_copy(src_hbm.at[c].at[pl.ds(hbase, HALF)], src_v)
            pltpu.sync_copy(dst_hbm.at[pl.ds(hbase, HALF)], dst_v)

            @pl.loop(0, HALF // 2)
            def _(t):
                j0 = 2 * t

                @pl.when(t > 0)
                def _():
                    pltpu.make_async_copy(
                        rows0, acc_sh.at[dst_v.at[j0 - 2]], sem_s0).wait()

                pltpu.async_copy(tab_hbm.at[src_v.at[j0]], rows0,
                                 sem_g0).wait()
                pltpu.async_copy(rows0, acc_sh.at[dst_v.at[j0]], sem_s0,
                                 add=True)

                @pl.when(t > 0)
                def _():
                    pltpu.make_async_copy(
                        rows1, acc_sh.at[dst_v.at[j0 - 1]], sem_s1).wait()

                pltpu.async_copy(tab_hbm.at[src_v.at[j0 + 1]], rows1,
                                 sem_g1).wait()
                pltpu.async_copy(rows1, acc_sh.at[dst_v.at[j0 + 1]], sem_s1,
                                 add=True)

            pltpu.make_async_copy(
                rows0, acc_sh.at[dst_v.at[HALF - 2]], sem_s0).wait()
            pltpu.make_async_copy(
                rows1, acc_sh.at[dst_v.at[HALF - 1]], sem_s1).wait()

        plsc.subcore_barrier()
        ob = s * CB
        pltpu.sync_copy(acc_sh.at[pl.ds(ob, CB)],
                        out_hbm.at[c].at[pl.ds(ob, CB)])

        @pl.when(s == NW - 1)
        def _():
            pltpu.sync_copy(acc_sh.at[pl.ds(NW * CB, CTAIL)],
                            out_hbm.at[c].at[pl.ds(NW * CB, CTAIL)])

    acc = agg_k(src3d, dst2d, h2s, zeros_b)

    b2 = jnp.broadcast_to(b.reshape(1, D), (8, D))

    def fin_body(acc_ref, h2_ref, cnt_ref, b_ref, o_ref):
        dis = lax.rsqrt(cnt_ref[...].astype(jnp.float32) + 1.0)
        o_ref[...] = (acc_ref[0] + h2_ref[...]) * dis + b_ref[0:1, :]

    out = pl.pallas_call(
        fin_body,
        grid=(N // BM, 2),
        in_specs=[
            pl.BlockSpec((1, BM, Dh), lambda r, c: (c, r, 0)),
            pl.BlockSpec((BM, Dh), lambda r, c: (c * (N // BM) + r, 0)),
            pl.BlockSpec((BM, 1), lambda r, c: (r, 0)),
            pl.BlockSpec((8, Dh), lambda r, c: (0, c)),
        ],
        out_specs=pl.BlockSpec((BM, Dh), lambda r, c: (r, c)),
        out_shape=jax.ShapeDtypeStruct((N, D), jnp.float32),
    )(acc, h2s, cnt_col, b2)
    return out


# speedup vs baseline: 1.1596x; 1.1596x over previous
import dataclasses
import functools

import jax
import jax.numpy as jnp
from jax import lax
from jax.experimental import pallas as pl
from jax.experimental.pallas import tpu as pltpu
from jax.experimental.pallas import tpu_sc as plsc


def kernel(x, edge_index, W, b):
    N, D = x.shape
    E = edge_index.shape[1]
    Dh = D // 2

    CH = 128
    NW = 16
    PAD_UNIT = 2 * NW * CH
    E_pad = ((E + PAD_UNIT - 1) // PAD_UNIT) * PAD_UNIT
    ROWS = E_pad // CH
    RPC = ROWS // 2
    RPS_A = RPC // NW
    RPS_B = ROWS // NW

    NT = N + NW
    ZB = (NT // (8 * NW)) * 8
    ZTAIL = NT - NW * ZB
    CB = (N // (8 * NW)) * 8
    CTAIL = N - NW * CB
    assert D % 2 == 0 and N % 8 == 0 and ZTAIL <= CH and CTAIL <= CH

    src = edge_index[0].astype(jnp.int32)
    dst = edge_index[1].astype(jnp.int32)
    pad = E_pad - E
    src_p = jnp.concatenate([src, jnp.zeros((pad,), jnp.int32)])
    dst_p = jnp.concatenate([dst, jnp.full((pad,), N, jnp.int32)])
    dst2d = dst_p.reshape(ROWS, CH)
    src2d = src_p.reshape(ROWS, CH)
    src3d = jnp.stack([src2d, src2d + N])

    zeros_b = jnp.zeros((CH, Dh), jnp.float32)

    mesh = plsc.VectorSubcoreMesh(core_axis_name="c", subcore_axis_name="s")


    EPW = E_pad // (2 * NW)
    assert EPW % 16 == 0 and N % 16 == 0

    @functools.partial(
        pl.kernel,
        out_type=jax.ShapeDtypeStruct((2, NW, 1, N), jnp.int32),
        mesh=mesh,
        scratch_types=[
            pltpu.VMEM((EPW,), jnp.int32),
            pltpu.VMEM((NT,), jnp.int32),
            pltpu.SemaphoreType.DMA,
        ],
        compiler_params=dataclasses.replace(
            pltpu.CompilerParams(), needs_layout_passes=False),
    )
    def degree_k(dst1d_hbm, deg_hbm, idx_v, hist_v, sem):
        c = lax.axis_index("c")
        s = lax.axis_index("s")
        w = c * NW + s
        pltpu.sync_copy(dst1d_hbm.at[pl.ds(w * EPW, EPW)], idx_v)

        @pl.loop(0, NT, step=16)
        def _(i):
            hist_v[pl.ds(i, 16)] = jnp.zeros((16,), jnp.int32)

        ones16 = jnp.ones((16,), jnp.int32)

        @pl.loop(0, EPW, step=16)
        def _(i):
            plsc.addupdate_scatter(hist_v, [idx_v[pl.ds(i, 16)]], ones16)

        pltpu.sync_copy(hist_v.at[pl.ds(0, N)], deg_hbm.at[c, s, 0])

    deg_hists = degree_k(dst_p)

    def degsum_body(d_ref, o_ref):
        o_ref[...] = jnp.sum(d_ref[...], axis=(0, 1))

    cnt_row = pl.pallas_call(
        degsum_body,
        grid=(1,),
        in_specs=[pl.BlockSpec((2, NW, 1, N), lambda i: (0, 0, 0, 0))],
        out_specs=pl.BlockSpec((1, N), lambda i: (0, 0)),
        out_shape=jax.ShapeDtypeStruct((1, N), jnp.int32),
    )(deg_hists)
    cnt_col = cnt_row.reshape(N, 1)

    BM = 1000

    def mm_body(x_ref, w_ref, cnt_ref, o_ref):
        dis = lax.rsqrt(cnt_ref[...].astype(jnp.float32) + 1.0)
        o_ref[...] = jnp.dot(x_ref[...], w_ref[...],
                             preferred_element_type=jnp.float32) * dis

    h2s = pl.pallas_call(
        mm_body,
        grid=(N // BM, 2),
        in_specs=[pl.BlockSpec((BM, D), lambda r, c: (r, 0)),
                  pl.BlockSpec((D, Dh), lambda r, c: (0, c)),
                  pl.BlockSpec((BM, 1), lambda r, c: (r, 0))],
        out_specs=pl.BlockSpec((BM, Dh), lambda r, c: (c * (N // BM) + r, 0)),
        out_shape=jax.ShapeDtypeStruct((2 * N, Dh), jnp.float32),
    )(x, W, cnt_col)

    HALF = RPS_B // 2
    assert HALF % 2 == 0

    @functools.partial(
        pl.kernel,
        out_type=jax.ShapeDtypeStruct((2, N, Dh), jnp.float32),
        mesh=mesh,
        scratch_types=[
            pltpu.VMEM((HALF, CH), jnp.int32),
            pltpu.VMEM((HALF, CH), jnp.int32),
            pltpu.VMEM((CH, Dh), jnp.float32),
            pltpu.VMEM((CH, Dh), jnp.float32),
            pltpu.VMEM_SHARED((NT, Dh), jnp.float32),
            pltpu.SemaphoreType.DMA,
            pltpu.SemaphoreType.DMA,
            pltpu.SemaphoreType.DMA,
            pltpu.SemaphoreType.DMA,
        ],
    )
    def agg_k(src_hbm, dst_hbm, tab_hbm, zeros_hbm, out_hbm,
              src_v, dst_v, rows0, rows1, acc_sh,
              sem_g0, sem_g1, sem_s0, sem_s1):
        c = lax.axis_index("c")
        s = lax.axis_index("s")
        pltpu.sync_copy(zeros_hbm, rows0)
        zbase = s * ZB
        nfull = ZB // CH

        @pl.loop(0, nfull)
        def _(i):
            pltpu.sync_copy(rows0, acc_sh.at[pl.ds(zbase + i * CH, CH)])

        rem = ZB - nfull * CH
        pltpu.sync_copy(rows0.at[pl.ds(0, rem)],
                        acc_sh.at[pl.ds(zbase + nfull * CH, rem)])

        @pl.when(s == NW - 1)
        def _():
            pltpu.sync_copy(rows0.at[pl.ds(0, ZTAIL)],
                            acc_sh.at[pl.ds(NW * ZB, ZTAIL)])

        plsc.subcore_barrier()
        row0 = s * RPS_B

        for half in range(2):
            hbase = row0 + half * HALF
            pltpu.sync_copy(src_hbm.at[c].at[pl.ds(hbase, HALF)], src_v)
            pltpu.sync_copy(dst_hbm.at[pl.ds(hbase, HALF)], dst_v)

            @pl.loop(0, HALF // 2)
            def _(t):
                j0 = 2 * t

                @pl.when(t > 0)
                def _():
                    pltpu.make_async_copy(
                        rows0, acc_sh.at[dst_v.at[j0 - 2]], sem_s0).wait()

                pltpu.async_copy(tab_hbm.at[src_v.at[j0]], rows0,
                                 sem_g0).wait()
                pltpu.async_copy(rows0, acc_sh.at[dst_v.at[j0]], sem_s0,
                                 add=True)

                @pl.when(t > 0)
                def _():
                    pltpu.make_async_copy(
                        rows1, acc_sh.at[dst_v.at[j0 - 1]], sem_s1).wait()

                pltpu.async_copy(tab_hbm.at[src_v.at[j0 + 1]], rows1,
                                 sem_g1).wait()
                pltpu.async_copy(rows1, acc_sh.at[dst_v.at[j0 + 1]], sem_s1,
                                 add=True)

            pltpu.make_async_copy(
                rows0, acc_sh.at[dst_v.at[HALF - 2]], sem_s0).wait()
            pltpu.make_async_copy(
                rows1, acc_sh.at[dst_v.at[HALF - 1]], sem_s1).wait()

        plsc.subcore_barrier()
        ob = s * CB
        pltpu.sync_copy(acc_sh.at[pl.ds(ob, CB)],
                        out_hbm.at[c].at[pl.ds(ob, CB)])

        @pl.when(s == NW - 1)
        def _():
            pltpu.sync_copy(acc_sh.at[pl.ds(NW * CB, CTAIL)],
                            out_hbm.at[c].at[pl.ds(NW * CB, CTAIL)])

    acc = agg_k(src3d, dst2d, h2s, zeros_b)

    b2 = jnp.broadcast_to(b.reshape(1, D), (8, D))

    def fin_body(acc_ref, h2_ref, cnt_ref, b_ref, o_ref):
        dis = lax.rsqrt(cnt_ref[...].astype(jnp.float32) + 1.0)
        o_ref[...] = (acc_ref[0] + h2_ref[...]) * dis + b_ref[0:1, :]

    out = pl.pallas_call(
        fin_body,
        grid=(N // BM, 2),
        in_specs=[
            pl.BlockSpec((1, BM, Dh), lambda r, c: (c, r, 0)),
            pl.BlockSpec((BM, Dh), lambda r, c: (c * (N // BM) + r, 0)),
            pl.BlockSpec((BM, 1), lambda r, c: (r, 0)),
            pl.BlockSpec((8, Dh), lambda r, c: (0, c)),
        ],
        out_specs=pl.BlockSpec((BM, Dh), lambda r, c: (r, c)),
        out_shape=jax.ShapeDtypeStruct((N, D), jnp.float32),
    )(acc, h2s, cnt_col, b2)
    return out


# speedup vs baseline: 1.2001x; 1.0349x over previous
import dataclasses
import functools

import jax
import jax.numpy as jnp
from jax import lax
from jax.experimental import pallas as pl
from jax.experimental.pallas import tpu as pltpu
from jax.experimental.pallas import tpu_sc as plsc


def kernel(x, edge_index, W, b):
    N, D = x.shape
    E = edge_index.shape[1]
    Dh = D // 2

    CH = 128
    NW = 16
    PAD_UNIT = 2 * NW * CH
    E_pad = ((E + PAD_UNIT - 1) // PAD_UNIT) * PAD_UNIT
    ROWS = E_pad // CH
    RPS_B = ROWS // NW

    NT = N + NW
    ZB = (NT // (8 * NW)) * 8
    ZTAIL = NT - NW * ZB
    CB = (N // (8 * NW)) * 8
    CTAIL = N - NW * CB
    assert D % 2 == 0 and N % 8 == 0 and ZTAIL <= CH and CTAIL <= CH

    src = edge_index[0].astype(jnp.int32)
    dst = edge_index[1].astype(jnp.int32)
    pad = E_pad - E
    src_p = jnp.concatenate([src, jnp.zeros((pad,), jnp.int32)])
    dst_p = jnp.concatenate([dst, jnp.full((pad,), N, jnp.int32)])
    dst2d = dst_p.reshape(ROWS, CH)
    src2d = src_p.reshape(ROWS, CH)
    src3d = jnp.stack([src2d, src2d + N])

    zeros_b = jnp.zeros((CH, Dh), jnp.float32)

    mesh = plsc.VectorSubcoreMesh(core_axis_name="c", subcore_axis_name="s")


    EPW = E_pad // (2 * NW)
    assert EPW % 16 == 0 and N % 16 == 0

    @functools.partial(
        pl.kernel,
        out_type=jax.ShapeDtypeStruct((2, NW, 1, N), jnp.int32),
        mesh=mesh,
        scratch_types=[
            pltpu.VMEM((EPW,), jnp.int32),
            pltpu.VMEM((NT,), jnp.int32),
            pltpu.SemaphoreType.DMA,
        ],
        compiler_params=dataclasses.replace(
            pltpu.CompilerParams(), needs_layout_passes=False),
    )
    def degree_k(dst1d_hbm, deg_hbm, idx_v, hist_v, sem):
        c = lax.axis_index("c")
        s = lax.axis_index("s")
        w = c * NW + s
        pltpu.sync_copy(dst1d_hbm.at[pl.ds(w * EPW, EPW)], idx_v)

        @pl.loop(0, NT, step=16)
        def _(i):
            hist_v[pl.ds(i, 16)] = jnp.zeros((16,), jnp.int32)

        ones16 = jnp.ones((16,), jnp.int32)

        @pl.loop(0, EPW, step=16)
        def _(i):
            plsc.addupdate_scatter(hist_v, [idx_v[pl.ds(i, 16)]], ones16)

        pltpu.sync_copy(hist_v.at[pl.ds(0, N)], deg_hbm.at[c, s, 0])

    deg_hists = degree_k(dst_p)

    def degsum_body(d_ref, o_ref):
        o_ref[...] = jnp.sum(d_ref[...], axis=(0, 1))

    cnt_row = pl.pallas_call(
        degsum_body,
        grid=(1,),
        in_specs=[pl.BlockSpec((2, NW, 1, N), lambda i: (0, 0, 0, 0))],
        out_specs=pl.BlockSpec((1, N), lambda i: (0, 0)),
        out_shape=jax.ShapeDtypeStruct((1, N), jnp.int32),
    )(deg_hists)
    cnt_col = cnt_row.reshape(N, 1)

    BM = 2000

    def mm_body(x_ref, w_ref, cnt_ref, o_ref):
        dis = lax.rsqrt(cnt_ref[...].astype(jnp.float32) + 1.0)
        o_ref[...] = jnp.dot(x_ref[...], w_ref[...],
                             preferred_element_type=jnp.float32) * dis

    h2s = pl.pallas_call(
        mm_body,
        grid=(N // BM, 2),
        in_specs=[pl.BlockSpec((BM, D), lambda r, c: (r, 0)),
                  pl.BlockSpec((D, Dh), lambda r, c: (0, c)),
                  pl.BlockSpec((BM, 1), lambda r, c: (r, 0))],
        out_specs=pl.BlockSpec((BM, Dh), lambda r, c: (c * (N // BM) + r, 0)),
        out_shape=jax.ShapeDtypeStruct((2 * N, Dh), jnp.float32),
    )(x, W, cnt_col)

    HALF = RPS_B // 2
    assert HALF % 2 == 0

    @functools.partial(
        pl.kernel,
        out_type=jax.ShapeDtypeStruct((2, N, Dh), jnp.float32),
        mesh=mesh,
        scratch_types=[
            pltpu.VMEM((HALF, CH), jnp.int32),
            pltpu.VMEM((HALF, CH), jnp.int32),
            pltpu.VMEM((CH, Dh), jnp.float32),
            pltpu.VMEM((CH, Dh), jnp.float32),
            pltpu.VMEM_SHARED((NT, Dh), jnp.float32),
            pltpu.SemaphoreType.DMA,
            pltpu.SemaphoreType.DMA,
            pltpu.SemaphoreType.DMA,
            pltpu.SemaphoreType.DMA,
        ],
    )
    def agg_k(src_hbm, dst_hbm, tab_hbm, zeros_hbm, out_hbm,
              src_v, dst_v, rows0, rows1, acc_sh,
              sem_g0, sem_g1, sem_s0, sem_s1):
        c = lax.axis_index("c")
        s = lax.axis_index("s")
        pltpu.sync_copy(zeros_hbm, rows0)
        zbase = s * ZB
        nfull = ZB // CH

        @pl.loop(0, nfull)
        def _(i):
            pltpu.sync_copy(rows0, acc_sh.at[pl.ds(zbase + i * CH, CH)])

        rem = ZB - nfull * CH
        pltpu.sync_copy(rows0.at[pl.ds(0, rem)],
                        acc_sh.at[pl.ds(zbase + nfull * CH, rem)])

        @pl.when(s == NW - 1)
        def _():
            pltpu.sync_copy(rows0.at[pl.ds(0, ZTAIL)],
                            acc_sh.at[pl.ds(NW * ZB, ZTAIL)])

        plsc.subcore_barrier()
        row0 = s * RPS_B

        for half in range(2):
            hbase = row0 + half * HALF
            pltpu.sync_copy(src_hbm.at[c].at[pl.ds(hbase, HALF)], src_v)
            pltpu.sync_copy(dst_hbm.at[pl.ds(hbase, HALF)], dst_v)

            @pl.loop(0, HALF // 2)
            def _(t):
                j0 = 2 * t

                @pl.when(t > 0)
                def _():
                    pltpu.make_async_copy(
                        rows0, acc_sh.at[dst_v.at[j0 - 2]], sem_s0).wait()

                pltpu.async_copy(tab_hbm.at[src_v.at[j0]], rows0,
                                 sem_g0).wait()
                pltpu.async_copy(rows0, acc_sh.at[dst_v.at[j0]], sem_s0,
                                 add=True)

                @pl.when(t > 0)
                def _():
                    pltpu.make_async_copy(
                        rows1, acc_sh.at[dst_v.at[j0 - 1]], sem_s1).wait()

                pltpu.async_copy(tab_hbm.at[src_v.at[j0 + 1]], rows1,
                                 sem_g1).wait()
                pltpu.async_copy(rows1, acc_sh.at[dst_v.at[j0 + 1]], sem_s1,
                                 add=True)

            pltpu.make_async_copy(
                rows0, acc_sh.at[dst_v.at[HALF - 2]], sem_s0).wait()
            pltpu.make_async_copy(
                rows1, acc_sh.at[dst_v.at[HALF - 1]], sem_s1).wait()

        plsc.subcore_barrier()
        ob = s * CB
        pltpu.sync_copy(acc_sh.at[pl.ds(ob, CB)],
                        out_hbm.at[c].at[pl.ds(ob, CB)])

        @pl.when(s == NW - 1)
        def _():
            pltpu.sync_copy(acc_sh.at[pl.ds(NW * CB, CTAIL)],
                            out_hbm.at[c].at[pl.ds(NW * CB, CTAIL)])

    acc = agg_k(src3d, dst2d, h2s, zeros_b)

    b2 = jnp.broadcast_to(b.reshape(1, D), (8, D))

    def fin_body(acc_ref, h2_ref, cnt_ref, b_ref, o_ref):
        dis = lax.rsqrt(cnt_ref[...].astype(jnp.float32) + 1.0)
        o_ref[...] = (acc_ref[0] + h2_ref[...]) * dis + b_ref[0:1, :]

    out = pl.pallas_call(
        fin_body,
        grid=(N // BM, 2),
        in_specs=[
            pl.BlockSpec((1, BM, Dh), lambda r, c: (c, r, 0)),
            pl.BlockSpec((BM, Dh), lambda r, c: (c * (N // BM) + r, 0)),
            pl.BlockSpec((BM, 1), lambda r, c: (r, 0)),
            pl.BlockSpec((8, Dh), lambda r, c: (0, c)),
        ],
        out_specs=pl.BlockSpec((BM, Dh), lambda r, c: (r, c)),
        out_shape=jax.ShapeDtypeStruct((N, D), jnp.float32),
    )(acc, h2s, cnt_col, b2)
    return out
